# Initial kernel scaffold; baseline (speedup 1.0000x reference)
import jax
import jax.numpy as jnp
from jax.experimental import pallas as pl


def kernel(inputs, selected_edges, hidden_con, hidden_uncon, rel_emb, ws, b, out_w, out_b):
    raise NotImplementedError("write your pallas kernel here")


# speedup vs baseline: 4.2343x; 4.2343x over previous
import jax
import jax.numpy as jnp
from jax import lax
from jax.experimental import pallas as pl
from jax.experimental.pallas import tpu as pltpu
from jax.experimental.pallas import tpu_sc as plsc

_N = 10000
_E = 320000
_D = 128
_NC = 2
_NS = 16
_NW = _NC * _NS
_C1 = 80
_EPW = _E // _NW
_C23 = 80
_EPT = _E // _NS

_mesh = plsc.VectorSubcoreMesh(core_axis_name="c", subcore_axis_name="s")


def _p1_body(vi_hbm, vj_hbm, rel_hbm, hc_hbm, hu_hbm, re_hbm, ws_hbm, b_hbm,
             ow_hbm, ob_hbm, exps_hbm,
             vi_v, vj_v, rel_v, a_v, au_v, r_v, c_v, cu_v,
             wsv, bv, owv, obv, acc_buf, ex_v, sem):
    cid = lax.axis_index("c")
    sid = lax.axis_index("s")
    wid = sid * _NC + cid
    pltpu.sync_copy(ws_hbm, wsv)
    pltpu.sync_copy(b_hbm, bv)
    pltpu.sync_copy(ow_hbm, owv)
    pltpu.sync_copy(ob_hbm, obv)
    base0 = wid * _EPW

    @pl.loop(0, _EPW // _C1)
    def _chunk(k):
        base = base0 + k * _C1
        pltpu.sync_copy(vi_hbm.at[pl.ds(base, _C1)], vi_v)
        pltpu.sync_copy(vj_hbm.at[pl.ds(base, _C1)], vj_v)
        pltpu.sync_copy(rel_hbm.at[pl.ds(base, _C1)], rel_v)
        cps = [pltpu.async_copy(hc_hbm.at[vi_v], a_v, sem),
               pltpu.async_copy(hu_hbm.at[vi_v], au_v, sem),
               pltpu.async_copy(re_hbm.at[rel_v], r_v, sem),
               pltpu.async_copy(hc_hbm.at[vj_v], c_v, sem),
               pltpu.async_copy(hu_hbm.at[vj_v], cu_v, sem)]
        for cp in cps:
            cp.wait()
        for g in range(_D // 16):
            dsl = pl.ds(g * 16, 16)
            w0 = wsv[0, dsl]
            w1 = wsv[1, dsl]
            w2 = wsv[2, dsl]
            w3 = wsv[3, dsl]
            w4 = wsv[4, dsl]
            w5 = wsv[5, dsl]
            w6 = wsv[6, dsl]
            w7 = wsv[7, dsl]
            bb = bv[dsl]
            ow = owv[dsl]
            ob = obv[dsl]

            def ebody(e, _, g=g, dsl=dsl, w0=w0, w1=w1, w2=w2, w3=w3, w4=w4,
                      w5=w5, w6=w6, w7=w7, bb=bb, ow=ow, ob=ob):
                a = a_v[e, dsl]
                au = au_v[e, dsl]
                r = r_v[e, dsl]
                c = c_v[e, dsl]
                cu = cu_v[e, dsl]
                p = a * (w0 + w1 * r) + au * (w4 + w5 * r)
                q = a * (w2 + w3 * r) + au * (w6 + w7 * r)
                t = c * p + cu * q + bb
                t = jnp.maximum(t, 0.0) * ow + ob
                sl = pl.ds(e * 16, 16)
                if g == 0:
                    acc_buf[sl] = t
                else:
                    acc_buf[sl] = acc_buf[sl] + t
                return ()

            lax.fori_loop(0, _C1, ebody, ())
        for g in range(_C1 // 16):
            rows16 = (lax.iota(jnp.int32, 16) + (g * 16)) * 16
            tot = jnp.zeros((16,), jnp.float32)
            for j in range(16):
                tot = tot + plsc.load_gather(acc_buf, [rows16 + j])
            ex_v[pl.ds(g * 16, 16)] = jnp.exp(tot)
        pltpu.sync_copy(ex_v, exps_hbm.at[pl.ds(base, _C1)])


def _p2_body(exps_hbm, ivi_hbm, zeros_hbm, denom_hbm, idx_v, val_v, shared, sem):
    cid = lax.axis_index("c")
    sid = lax.axis_index("s")

    @pl.when(cid == 0)
    def _():
        @pl.when(sid == 0)
        def _zero():
            pltpu.sync_copy(zeros_hbm, shared)

        plsc.subcore_barrier()
        base0 = sid * _EPT

        @pl.loop(0, _EPT // _C23)
        def _chunk(k):
            base = base0 + k * _C23
            pltpu.sync_copy(ivi_hbm.at[pl.ds(base, _C23)], idx_v)
            pltpu.sync_copy(exps_hbm.at[pl.ds(base, _C23)], val_v)
            pltpu.sync_copy(val_v, shared.at[idx_v], add=True)

        plsc.subcore_barrier()

        @pl.when(sid == 0)
        def _out():
            pltpu.sync_copy(shared, denom_hbm)


def _p3_body(exps_hbm, ivi_hbm, vi_hbm, vj_hbm, denom_hbm, inp_hbm, zeros_hbm,
             out_hbm, ivi_v, vi_v, vj_v, ex_v, den_v, inp_v, att_v, shared, sem):
    cid = lax.axis_index("c")
    sid = lax.axis_index("s")

    @pl.when(cid == 0)
    def _():
        @pl.when(sid == 0)
        def _zero():
            pltpu.sync_copy(zeros_hbm, shared)

        plsc.subcore_barrier()
        base0 = sid * _EPT

        @pl.loop(0, _EPT // _C23)
        def _chunk(k):
            base = base0 + k * _C23
            pltpu.sync_copy(ivi_hbm.at[pl.ds(base, _C23)], ivi_v)
            pltpu.sync_copy(vi_hbm.at[pl.ds(base, _C23)], vi_v)
            pltpu.sync_copy(vj_hbm.at[pl.ds(base, _C23)], vj_v)
            pltpu.sync_copy(exps_hbm.at[pl.ds(base, _C23)], ex_v)
            cps = [pltpu.async_copy(denom_hbm.at[ivi_v], den_v, sem),
                   pltpu.async_copy(inp_hbm.at[vi_v], inp_v, sem)]
            for cp in cps:
                cp.wait()
            for g in range(_C23 // 16):
                s = pl.ds(g * 16, 16)
                att_v[s] = ex_v[s] * inp_v[s] / den_v[s]
            pltpu.sync_copy(att_v, shared.at[vj_v], add=True)

        plsc.subcore_barrier()

        @pl.when(sid == 0)
        def _out():
            pltpu.sync_copy(shared, out_hbm)


_phase1 = pl.kernel(
    _p1_body,
    out_type=jax.ShapeDtypeStruct((_E,), jnp.float32),
    mesh=_mesh,
    compiler_params=pltpu.CompilerParams(needs_layout_passes=False),
    scratch_types=[
        pltpu.VMEM((_C1,), jnp.int32),
        pltpu.VMEM((_C1,), jnp.int32),
        pltpu.VMEM((_C1,), jnp.int32),
        pltpu.VMEM((_C1, _D), jnp.float32),
        pltpu.VMEM((_C1, _D), jnp.float32),
        pltpu.VMEM((_C1, _D), jnp.float32),
        pltpu.VMEM((_C1, _D), jnp.float32),
        pltpu.VMEM((_C1, _D), jnp.float32),
        pltpu.VMEM((8, _D), jnp.float32),
        pltpu.VMEM((_D,), jnp.float32),
        pltpu.VMEM((_D,), jnp.float32),
        pltpu.VMEM((_D,), jnp.float32),
        pltpu.VMEM((_C1 * 16,), jnp.float32),
        pltpu.VMEM((_C1,), jnp.float32),
        pltpu.SemaphoreType.DMA,
    ],
)

_phase2 = pl.kernel(
    _p2_body,
    out_type=jax.ShapeDtypeStruct((_N,), jnp.float32),
    mesh=_mesh,
    compiler_params=pltpu.CompilerParams(needs_layout_passes=False),
    scratch_types=[
        pltpu.VMEM((_C23,), jnp.int32),
        pltpu.VMEM((_C23,), jnp.float32),
        pltpu.VMEM_SHARED((_N,), jnp.float32),
        pltpu.SemaphoreType.DMA,
    ],
)

_phase3 = pl.kernel(
    _p3_body,
    out_type=jax.ShapeDtypeStruct((_N,), jnp.float32),
    mesh=_mesh,
    compiler_params=pltpu.CompilerParams(needs_layout_passes=False),
    scratch_types=[
        pltpu.VMEM((_C23,), jnp.int32),
        pltpu.VMEM((_C23,), jnp.int32),
        pltpu.VMEM((_C23,), jnp.int32),
        pltpu.VMEM((_C23,), jnp.float32),
        pltpu.VMEM((_C23,), jnp.float32),
        pltpu.VMEM((_C23,), jnp.float32),
        pltpu.VMEM((_C23,), jnp.float32),
        pltpu.VMEM_SHARED((_N,), jnp.float32),
        pltpu.SemaphoreType.DMA,
    ],
)


def kernel(inputs, selected_edges, hidden_con, hidden_uncon, rel_emb, ws, b,
           out_w, out_b):
    vi = selected_edges[:, 1]
    vj = selected_edges[:, 2]
    rel = selected_edges[:, 3]
    ivi = selected_edges[:, 4]
    hc = hidden_con[0]
    hu = hidden_uncon[0]
    inp = inputs[0]
    zeros = jnp.zeros((_N,), jnp.float32)
    exps = _phase1(vi, vj, rel, hc, hu, rel_emb, ws, b, out_w, out_b)
    denom = _phase2(exps, ivi, zeros)
    out = _phase3(exps, ivi, vi, vj, denom, inp, zeros)
    return out.reshape(1, _N)
